# baseline (speedup 1.0000x reference)
import functools

import jax
import jax.numpy as jnp
from jax import lax
from jax.experimental import pallas as pl
from jax.experimental.pallas import tpu as pltpu
from jax.experimental.pallas import tpu_sc as plsc

H = 4
K = 256
D_IN = 384
D_HID = 256
D_LAT = 384
HD = D_LAT // H
B = 16384

BB = 512
S = 4
CS = B // S
NBLK = CS // BB

NC = 2
NS = 16
NW = NC * NS
CH = 256
HDP = 128


def _ln(x, g, b):
    mu = jnp.mean(x, axis=-1, keepdims=True)
    var = jnp.var(x, axis=-1, keepdims=True)
    return (x - mu) / jnp.sqrt(var + 1e-5) * g + b


def _enc_body(x_ref, W1_ref, b1_ref, g1_ref, be1_ref, W2_ref, b2_ref,
              g2_ref, be2_ref, R_ref, cb_ref, cb2t_ref, z_ref, idx_ref):
    x = x_ref[...]
    h = jax.nn.gelu(_ln(x @ W1_ref[...] + b1_ref[...], g1_ref[...], be1_ref[...]))
    z = _ln(h @ W2_ref[...] + b2_ref[...], g2_ref[...], be2_ref[...])
    z_ref[...] = z
    zrT = lax.dot_general(R_ref[...], z, (((0,), (1,)), ((), ())))
    iota_k = lax.broadcasted_iota(jnp.int32, (K, BB), 0).astype(jnp.float32)
    for hh in range(H):
        zhT = zrT[hh * HD:(hh + 1) * HD, :]
        aT = jnp.sum(zhT * zhT, axis=0, keepdims=True)
        pT = lax.dot_general(cb_ref[hh], zhT, (((1,), (0,)), ((), ())))
        dT = aT - 2.0 * pT + cb2t_ref[:, hh:hh + 1]
        m = jnp.min(dT, axis=0, keepdims=True)
        idxf = jnp.min(jnp.where(dT == m, iota_k, float(K)), axis=0)
        idx_ref[hh, :] = idxf.astype(jnp.int32)


def _dec_body(q_ref, RTp_ref, W3_ref, b3_ref, g3_ref, be3_ref, W4_ref, b4_ref,
              zq_ref, rec_ref):
    qp = q_ref[...]
    zq = lax.dot_general(qp.astype(jnp.bfloat16), RTp_ref[...],
                         (((1,), (0,)), ((), ())),
                         preferred_element_type=jnp.float32)
    zq_ref[...] = zq
    h2 = jax.nn.gelu(_ln(
        lax.dot_general(zq.astype(jnp.bfloat16), W3_ref[...],
                        (((1,), (0,)), ((), ())),
                        preferred_element_type=jnp.float32) + b3_ref[...],
        g3_ref[...], be3_ref[...]))
    rec_ref[...] = lax.dot_general(h2.astype(jnp.bfloat16), W4_ref[...],
                                   (((1,), (0,)), ((), ())),
                                   preferred_element_type=jnp.float32) + b4_ref[...]


def _full(shape):
    return pl.BlockSpec(shape, lambda i: tuple(0 for _ in shape))


def _encoder_call(x, W1, b1, g1, be1, W2, b2, g2, be2, R, codebook, cb2t, s):
    return pl.pallas_call(
        _enc_body,
        grid=(NBLK,),
        in_specs=[
            pl.BlockSpec((BB, D_IN), lambda i: (i + s * NBLK, 0)),
            _full((D_IN, D_HID)), _full((D_HID,)), _full((D_HID,)), _full((D_HID,)),
            _full((D_HID, D_LAT)), _full((D_LAT,)), _full((D_LAT,)), _full((D_LAT,)),
            _full((D_LAT, D_LAT)),
            _full((H, K, HD)),
            _full((K, H)),
        ],
        out_specs=[
            pl.BlockSpec((BB, D_LAT), lambda i: (i, 0)),
            pl.BlockSpec((H, BB), lambda i: (0, i)),
        ],
        out_shape=[
            jax.ShapeDtypeStruct((CS, D_LAT), jnp.float32),
            jax.ShapeDtypeStruct((H, CS), jnp.int32),
        ],
        compiler_params=pltpu.CompilerParams(
            dimension_semantics=("parallel",)),
    )(x, W1, b1, g1, be1, W2, b2, g2, be2, R, codebook, cb2t)


def _decoder_call(qcat, RTp, W3, b3, g3, be3, W4, b4):
    return pl.pallas_call(
        _dec_body,
        grid=(NBLK,),
        in_specs=[
            pl.BlockSpec((BB, H * HDP), lambda i: (i, 0)),
            _full((H * HDP, D_LAT)),
            _full((D_LAT, D_HID)), _full((D_HID,)), _full((D_HID,)), _full((D_HID,)),
            _full((D_HID, D_IN)), _full((D_IN,)),
        ],
        out_specs=[
            pl.BlockSpec((BB, D_LAT), lambda i: (i, 0)),
            pl.BlockSpec((BB, D_IN), lambda i: (i, 0)),
        ],
        out_shape=[
            jax.ShapeDtypeStruct((CS, D_LAT), jnp.float32),
            jax.ShapeDtypeStruct((CS, D_IN), jnp.float32),
        ],
        compiler_params=pltpu.CompilerParams(
            dimension_semantics=("parallel",)),
    )(qcat, RTp, W3, b3, g3, be3, W4, b4)


def _sc_gather(table, idx_flat):
    n_rows = idx_flat.shape[0]
    BPW = n_rows // NW
    mesh = plsc.VectorSubcoreMesh(core_axis_name="c", subcore_axis_name="s")

    @functools.partial(
        pl.kernel,
        mesh=mesh,
        out_type=jax.ShapeDtypeStruct((n_rows, HDP), jnp.float32),
        scratch_types=[
            pltpu.VMEM((CH,), jnp.int32),
            pltpu.VMEM((CH,), jnp.int32),
            pltpu.VMEM((CH, HDP), jnp.float32),
            pltpu.VMEM((CH, HDP), jnp.float32),
            pltpu.SemaphoreType.DMA,
            pltpu.SemaphoreType.DMA,
        ],
    )
    def gather_k(table_hbm, idx_hbm, out_hbm, idx0, idx1, rows0, rows1,
                 sem0, sem1):
        wid = lax.axis_index("s") * NC + lax.axis_index("c")
        base = wid * BPW
        idxb = (idx0, idx1)
        rows = (rows0, rows1)
        sems = (sem0, sem1)
        num = BPW // CH
        cps = [None, None]
        for c in range(num):
            bu = c % 2
            pltpu.sync_copy(idx_hbm.at[pl.ds(base + c * CH, CH)], idxb[bu])
            cps[bu] = pltpu.async_copy(table_hbm.at[idxb[bu]], rows[bu],
                                       sems[bu])
            if c > 0:
                cps[1 - bu].wait()
                pltpu.sync_copy(rows[1 - bu],
                                out_hbm.at[pl.ds(base + (c - 1) * CH, CH)])
        last = (num - 1) % 2
        cps[last].wait()
        pltpu.sync_copy(rows[last], out_hbm.at[pl.ds(base + (num - 1) * CH, CH)])

    return gather_k(table, idx_flat)


def kernel(x, W1, b1, g1, be1, W2, b2, g2, be2, R, codebook, W3, b3, g3, be3,
           W4, b4):
    cb2t = jnp.sum(codebook * codebook, axis=-1).T
    table = jnp.pad(codebook.reshape(H * K, HD), ((0, 0), (0, HDP - HD)))
    RTp = jnp.pad(R.T.reshape(H, HD, D_LAT), ((0, 0), (0, HDP - HD), (0, 0))
                  ).reshape(H * HDP, D_LAT).astype(jnp.bfloat16)
    W3b = W3.astype(jnp.bfloat16)
    W4b = W4.astype(jnp.bfloat16)
    offs = (K * jnp.arange(H, dtype=jnp.int32))[:, None]

    zs, idxs, qs = [], [], []
    for s in range(S):
        z_s, idx_s = _encoder_call(x, W1, b1, g1, be1, W2, b2, g2, be2, R,
                                   codebook, cb2t, s)
        idx_flat = (idx_s + offs).T.reshape(-1)
        qs.append(_sc_gather(table, idx_flat))
        zs.append(z_s)
        idxs.append(idx_s)
    outs = [_decoder_call(q.reshape(CS, H * HDP), RTp, W3b, b3, g3, be3,
                          W4b, b4) for q in qs]
    reconstructed = jnp.concatenate([o[1] for o in outs], axis=0)
    z_q = jnp.concatenate([o[0] for o in outs], axis=0)
    z = jnp.concatenate(zs, axis=0)
    indices = jnp.concatenate(idxs, axis=1).T
    return (reconstructed, indices, z, z_q)


# speedup vs baseline: 1.2874x; 1.2874x over previous
import functools

import jax
import jax.numpy as jnp
from jax import lax
from jax.experimental import pallas as pl
from jax.experimental.pallas import tpu as pltpu
from jax.experimental.pallas import tpu_sc as plsc

H = 4
K = 256
D_IN = 384
D_HID = 256
D_LAT = 384
HD = D_LAT // H
B = 16384

BB = 512
S = 1
CS = B // S
NBLK = CS // BB

NC = 2
NS = 16
NW = NC * NS
CH = 256
HDP = 128


def _ln(x, g, b):
    mu = jnp.mean(x, axis=-1, keepdims=True)
    var = jnp.var(x, axis=-1, keepdims=True)
    return (x - mu) / jnp.sqrt(var + 1e-5) * g + b


def _enc_body(x_ref, W1_ref, b1_ref, g1_ref, be1_ref, W2_ref, b2_ref,
              g2_ref, be2_ref, R_ref, cb_ref, cb2t_ref, z_ref, idx_ref):
    x = x_ref[...]
    h = jax.nn.gelu(_ln(x @ W1_ref[...] + b1_ref[...], g1_ref[...], be1_ref[...]))
    z = _ln(h @ W2_ref[...] + b2_ref[...], g2_ref[...], be2_ref[...])
    z_ref[...] = z
    zrT = lax.dot_general(R_ref[...], z, (((0,), (1,)), ((), ())))
    iota_k = lax.broadcasted_iota(jnp.int32, (K, BB), 0).astype(jnp.float32)
    for hh in range(H):
        zhT = zrT[hh * HD:(hh + 1) * HD, :]
        aT = jnp.sum(zhT * zhT, axis=0, keepdims=True)
        pT = lax.dot_general(cb_ref[hh], zhT, (((1,), (0,)), ((), ())))
        dT = aT - 2.0 * pT + cb2t_ref[:, hh:hh + 1]
        m = jnp.min(dT, axis=0, keepdims=True)
        idxf = jnp.min(jnp.where(dT == m, iota_k, float(K)), axis=0)
        idx_ref[hh, :] = idxf.astype(jnp.int32)


def _dec_body(q_ref, RTp_ref, W3_ref, b3_ref, g3_ref, be3_ref, W4_ref, b4_ref,
              zq_ref, rec_ref):
    qp = q_ref[...].reshape(BB, H * HDP)
    zq = lax.dot_general(qp.astype(jnp.bfloat16), RTp_ref[...],
                         (((1,), (0,)), ((), ())),
                         preferred_element_type=jnp.float32)
    zq_ref[...] = zq
    h2 = jax.nn.gelu(_ln(
        lax.dot_general(zq.astype(jnp.bfloat16), W3_ref[...],
                        (((1,), (0,)), ((), ())),
                        preferred_element_type=jnp.float32) + b3_ref[...],
        g3_ref[...], be3_ref[...]))
    rec_ref[...] = lax.dot_general(h2.astype(jnp.bfloat16), W4_ref[...],
                                   (((1,), (0,)), ((), ())),
                                   preferred_element_type=jnp.float32) + b4_ref[...]


def _full(shape):
    return pl.BlockSpec(shape, lambda i: tuple(0 for _ in shape))


def _encoder_call(x, W1, b1, g1, be1, W2, b2, g2, be2, R, codebook, cb2t, s):
    return pl.pallas_call(
        _enc_body,
        grid=(NBLK,),
        in_specs=[
            pl.BlockSpec((BB, D_IN), lambda i: (i + s * NBLK, 0)),
            _full((D_IN, D_HID)), _full((D_HID,)), _full((D_HID,)), _full((D_HID,)),
            _full((D_HID, D_LAT)), _full((D_LAT,)), _full((D_LAT,)), _full((D_LAT,)),
            _full((D_LAT, D_LAT)),
            _full((H, K, HD)),
            _full((K, H)),
        ],
        out_specs=[
            pl.BlockSpec((BB, D_LAT), lambda i: (i, 0)),
            pl.BlockSpec((H, BB), lambda i: (0, i)),
        ],
        out_shape=[
            jax.ShapeDtypeStruct((CS, D_LAT), jnp.float32),
            jax.ShapeDtypeStruct((H, CS), jnp.int32),
        ],
        compiler_params=pltpu.CompilerParams(
            dimension_semantics=("parallel",)),
    )(x, W1, b1, g1, be1, W2, b2, g2, be2, R, codebook, cb2t)


def _decoder_call(qcat, RTp, W3, b3, g3, be3, W4, b4):
    return pl.pallas_call(
        _dec_body,
        grid=(NBLK,),
        in_specs=[
            pl.BlockSpec((BB * H, HDP), lambda i: (i, 0)),
            _full((H * HDP, D_LAT)),
            _full((D_LAT, D_HID)), _full((D_HID,)), _full((D_HID,)), _full((D_HID,)),
            _full((D_HID, D_IN)), _full((D_IN,)),
        ],
        out_specs=[
            pl.BlockSpec((BB, D_LAT), lambda i: (i, 0)),
            pl.BlockSpec((BB, D_IN), lambda i: (i, 0)),
        ],
        out_shape=[
            jax.ShapeDtypeStruct((CS, D_LAT), jnp.float32),
            jax.ShapeDtypeStruct((CS, D_IN), jnp.float32),
        ],
        compiler_params=pltpu.CompilerParams(
            dimension_semantics=("parallel",)),
    )(qcat, RTp, W3, b3, g3, be3, W4, b4)


def _sc_gather(table, idx_flat):
    n_rows = idx_flat.shape[0]
    BPW = n_rows // NW
    mesh = plsc.VectorSubcoreMesh(core_axis_name="c", subcore_axis_name="s")

    @functools.partial(
        pl.kernel,
        mesh=mesh,
        out_type=jax.ShapeDtypeStruct((n_rows, HDP), jnp.float32),
        scratch_types=[
            pltpu.VMEM((CH,), jnp.int32),
            pltpu.VMEM((CH,), jnp.int32),
            pltpu.VMEM((CH, HDP), jnp.float32),
            pltpu.VMEM((CH, HDP), jnp.float32),
            pltpu.SemaphoreType.DMA,
            pltpu.SemaphoreType.DMA,
        ],
    )
    def gather_k(table_hbm, idx_hbm, out_hbm, idx0, idx1, rows0, rows1,
                 sem0, sem1):
        wid = lax.axis_index("s") * NC + lax.axis_index("c")
        base = wid * BPW
        idxb = (idx0, idx1)
        rows = (rows0, rows1)
        sems = (sem0, sem1)
        num = BPW // CH
        cps = [None, None]
        for c in range(num):
            bu = c % 2
            pltpu.sync_copy(idx_hbm.at[pl.ds(base + c * CH, CH)], idxb[bu])
            cps[bu] = pltpu.async_copy(table_hbm.at[idxb[bu]], rows[bu],
                                       sems[bu])
            if c > 0:
                cps[1 - bu].wait()
                pltpu.sync_copy(rows[1 - bu],
                                out_hbm.at[pl.ds(base + (c - 1) * CH, CH)])
        last = (num - 1) % 2
        cps[last].wait()
        pltpu.sync_copy(rows[last], out_hbm.at[pl.ds(base + (num - 1) * CH, CH)])

    return gather_k(table, idx_flat)


def kernel(x, W1, b1, g1, be1, W2, b2, g2, be2, R, codebook, W3, b3, g3, be3,
           W4, b4):
    cb2t = jnp.sum(codebook * codebook, axis=-1).T
    table = jnp.pad(codebook.reshape(H * K, HD), ((0, 0), (0, HDP - HD)))
    RTp = jnp.pad(R.T.reshape(H, HD, D_LAT), ((0, 0), (0, HDP - HD), (0, 0))
                  ).reshape(H * HDP, D_LAT).astype(jnp.bfloat16)
    W3b = W3.astype(jnp.bfloat16)
    W4b = W4.astype(jnp.bfloat16)
    offs = (K * jnp.arange(H, dtype=jnp.int32))[:, None]

    zs, idxs, qs = [], [], []
    for s in range(S):
        z_s, idx_s = _encoder_call(x, W1, b1, g1, be1, W2, b2, g2, be2, R,
                                   codebook, cb2t, s)
        idx_flat = (idx_s + offs).T.reshape(-1)
        qs.append(_sc_gather(table, idx_flat))
        zs.append(z_s)
        idxs.append(idx_s)
    outs = [_decoder_call(q, RTp, W3b, b3, g3, be3, W4b, b4) for q in qs]
    reconstructed = jnp.concatenate([o[1] for o in outs], axis=0)
    z_q = jnp.concatenate([o[0] for o in outs], axis=0)
    z = jnp.concatenate(zs, axis=0)
    indices = jnp.concatenate(idxs, axis=1).T
    return (reconstructed, indices, z, z_q)


# speedup vs baseline: 1.3825x; 1.0739x over previous
import functools

import jax
import jax.numpy as jnp
from jax import lax
from jax.experimental import pallas as pl
from jax.experimental.pallas import tpu as pltpu
from jax.experimental.pallas import tpu_sc as plsc

H = 4
K = 256
D_IN = 384
D_HID = 256
D_LAT = 384
HD = D_LAT // H
B = 16384

BB = 512
S = 1
CS = B // S
NBLK = CS // BB

NC = 2
NS = 16
NW = NC * NS
CH = 256
HDP = 128


def _ln(x, g, b):
    mu = jnp.mean(x, axis=-1, keepdims=True)
    var = jnp.var(x, axis=-1, keepdims=True)
    return (x - mu) / jnp.sqrt(var + 1e-5) * g + b


def _enc_body(x_ref, W1_ref, b1_ref, g1_ref, be1_ref, W2_ref, b2_ref,
              g2_ref, be2_ref, R_ref, cb_ref, cb2t_ref, z_ref, idx_ref):
    x = x_ref[...]
    h = jax.nn.gelu(_ln(x @ W1_ref[...] + b1_ref[...], g1_ref[...], be1_ref[...]))
    z = _ln(h @ W2_ref[...] + b2_ref[...], g2_ref[...], be2_ref[...])
    z_ref[...] = z
    zrT = lax.dot_general(R_ref[...], z, (((0,), (1,)), ((), ())))
    iota_k = lax.broadcasted_iota(jnp.int32, (K, BB), 0).astype(jnp.float32)
    for hh in range(H):
        zhT = zrT[hh * HD:(hh + 1) * HD, :]
        aT = jnp.sum(zhT * zhT, axis=0, keepdims=True)
        pT = lax.dot_general(cb_ref[hh], zhT, (((1,), (0,)), ((), ())))
        dT = aT - 2.0 * pT + cb2t_ref[:, hh:hh + 1]
        m = jnp.min(dT, axis=0, keepdims=True)
        idxf = jnp.min(jnp.where(dT == m, iota_k, float(K)), axis=0)
        idx_ref[hh, :] = idxf.astype(jnp.int32)


def _dec_body(q_ref, RTp_ref, W3_ref, b3_ref, g3_ref, be3_ref, W4_ref, b4_ref,
              zq_ref, rec_ref):
    qp = q_ref[...].reshape(BB, H * HDP)
    zq = lax.dot_general(qp.astype(jnp.bfloat16), RTp_ref[...],
                         (((1,), (0,)), ((), ())),
                         preferred_element_type=jnp.float32)
    zq_ref[...] = zq
    h2 = jax.nn.gelu(_ln(
        lax.dot_general(zq.astype(jnp.bfloat16), W3_ref[...],
                        (((1,), (0,)), ((), ())),
                        preferred_element_type=jnp.float32) + b3_ref[...],
        g3_ref[...], be3_ref[...]))
    rec_ref[...] = lax.dot_general(h2.astype(jnp.bfloat16), W4_ref[...],
                                   (((1,), (0,)), ((), ())),
                                   preferred_element_type=jnp.float32) + b4_ref[...]


def _full(shape):
    return pl.BlockSpec(shape, lambda i: tuple(0 for _ in shape))


def _encoder_call(x, W1, b1, g1, be1, W2, b2, g2, be2, R, codebook, cb2t, s):
    return pl.pallas_call(
        _enc_body,
        grid=(NBLK,),
        in_specs=[
            pl.BlockSpec((BB, D_IN), lambda i: (i + s * NBLK, 0)),
            _full((D_IN, D_HID)), _full((D_HID,)), _full((D_HID,)), _full((D_HID,)),
            _full((D_HID, D_LAT)), _full((D_LAT,)), _full((D_LAT,)), _full((D_LAT,)),
            _full((D_LAT, D_LAT)),
            _full((H, K, HD)),
            _full((K, H)),
        ],
        out_specs=[
            pl.BlockSpec((BB, D_LAT), lambda i: (i, 0)),
            pl.BlockSpec((H, BB), lambda i: (0, i)),
        ],
        out_shape=[
            jax.ShapeDtypeStruct((CS, D_LAT), jnp.float32),
            jax.ShapeDtypeStruct((H, CS), jnp.int32),
        ],
        compiler_params=pltpu.CompilerParams(
            dimension_semantics=("parallel",)),
    )(x, W1, b1, g1, be1, W2, b2, g2, be2, R, codebook, cb2t)


def _decoder_call(qcat, RTp, W3, b3, g3, be3, W4, b4):
    return pl.pallas_call(
        _dec_body,
        grid=(NBLK,),
        in_specs=[
            pl.BlockSpec((BB * H, HDP), lambda i: (i, 0)),
            _full((H * HDP, D_LAT)),
            _full((D_LAT, D_HID)), _full((D_HID,)), _full((D_HID,)), _full((D_HID,)),
            _full((D_HID, D_IN)), _full((D_IN,)),
        ],
        out_specs=[
            pl.BlockSpec((BB, D_LAT), lambda i: (i, 0)),
            pl.BlockSpec((BB, D_IN), lambda i: (i, 0)),
        ],
        out_shape=[
            jax.ShapeDtypeStruct((CS, D_LAT), jnp.float32),
            jax.ShapeDtypeStruct((CS, D_IN), jnp.float32),
        ],
        compiler_params=pltpu.CompilerParams(
            dimension_semantics=("parallel",)),
    )(qcat, RTp, W3, b3, g3, be3, W4, b4)


TABW = H * K * HD
CH2 = 64


def _sc_gather(table_flat, idx_off):
    n_rows = idx_off.shape[0]
    BPW = n_rows // NW
    mesh = plsc.VectorSubcoreMesh(core_axis_name="c", subcore_axis_name="s")

    @functools.partial(
        pl.kernel,
        mesh=mesh,
        out_type=jax.ShapeDtypeStruct((n_rows, HDP), jnp.float32),
        scratch_types=[
            pltpu.VMEM((TABW,), jnp.float32),
            pltpu.VMEM((BPW,), jnp.int32),
            pltpu.VMEM((CH2, HDP), jnp.float32),
            pltpu.VMEM((CH2, HDP), jnp.float32),
            pltpu.SemaphoreType.DMA,
            pltpu.SemaphoreType.DMA,
        ],
    )
    def gather_k(table_hbm, idx_hbm, out_hbm, tab_v, idx_v, buf0, buf1,
                 sem0, sem1):
        wid = lax.axis_index("s") * NC + lax.axis_index("c")
        base = wid * BPW
        pltpu.sync_copy(table_hbm, tab_v)
        pltpu.sync_copy(idx_hbm.at[pl.ds(base, BPW)], idx_v)
        bufs = (buf0, buf1)
        sems = (sem0, sem1)
        z16 = jnp.zeros((16,), jnp.float32)
        for b in range(2):
            for r in range(CH2):
                bufs[b][r, pl.ds(HD, 16)] = z16
                bufs[b][r, pl.ds(HD + 16, 16)] = z16

        nsup = BPW // (2 * CH2)

        def super_body(c2, carry):
            for b in range(2):
                ch = c2 * 2 + b
                row0 = ch * CH2
                for gi in range(CH2 // 16):
                    g16 = idx_v[pl.ds(row0 + 16 * gi, 16)]
                    for l in range(16):
                        src = g16[l]
                        r = gi * 16 + l
                        for c6 in range(HD // 16):
                            bufs[b][r, pl.ds(16 * c6, 16)] = (
                                tab_v[pl.ds(src + 16 * c6, 16)])
                pltpu.async_copy(bufs[b],
                                 out_hbm.at[pl.ds(base + row0, CH2)],
                                 sems[b])
            for b in range(2):
                pltpu.make_async_copy(
                    bufs[b],
                    out_hbm.at[pl.ds(base + (c2 * 2 + b) * CH2, CH2)],
                    sems[b]).wait()
            return carry

        lax.fori_loop(0, nsup, super_body, 0)

    return gather_k(table_flat, idx_off)


def kernel(x, W1, b1, g1, be1, W2, b2, g2, be2, R, codebook, W3, b3, g3, be3,
           W4, b4):
    cb2t = jnp.sum(codebook * codebook, axis=-1).T
    table = codebook.reshape(-1)
    RTp = jnp.pad(R.T.reshape(H, HD, D_LAT), ((0, 0), (0, HDP - HD), (0, 0))
                  ).reshape(H * HDP, D_LAT).astype(jnp.bfloat16)
    W3b = W3.astype(jnp.bfloat16)
    W4b = W4.astype(jnp.bfloat16)
    offs = (K * jnp.arange(H, dtype=jnp.int32))[:, None]

    zs, idxs, qs = [], [], []
    for s in range(S):
        z_s, idx_s = _encoder_call(x, W1, b1, g1, be1, W2, b2, g2, be2, R,
                                   codebook, cb2t, s)
        idx_flat = ((idx_s + offs).T * HD).reshape(-1)
        qs.append(_sc_gather(table, idx_flat))
        zs.append(z_s)
        idxs.append(idx_s)
    outs = [_decoder_call(q, RTp, W3b, b3, g3, be3, W4b, b4) for q in qs]
    reconstructed = jnp.concatenate([o[1] for o in outs], axis=0)
    z_q = jnp.concatenate([o[0] for o in outs], axis=0)
    z = jnp.concatenate(zs, axis=0)
    indices = jnp.concatenate(idxs, axis=1).T
    return (reconstructed, indices, z, z_q)


# speedup vs baseline: 1.4670x; 1.0611x over previous
import functools

import jax
import jax.numpy as jnp
from jax import lax
from jax.experimental import pallas as pl
from jax.experimental.pallas import tpu as pltpu
from jax.experimental.pallas import tpu_sc as plsc

H = 4
K = 256
D_IN = 384
D_HID = 256
D_LAT = 384
HD = D_LAT // H
B = 16384

BB = 512
S = 1
CS = B // S
NBLK = CS // BB

NC = 2
NS = 16
NW = NC * NS
CH = 256
HDP = 128


def _ln(x, g, b):
    mu = jnp.mean(x, axis=-1, keepdims=True)
    var = jnp.var(x, axis=-1, keepdims=True)
    return (x - mu) / jnp.sqrt(var + 1e-5) * g + b


def _enc_body(x_ref, W1_ref, b1_ref, g1_ref, be1_ref, W2_ref, b2_ref,
              g2_ref, be2_ref, R_ref, cb_ref, cb2t_ref, z_ref, idx_ref):
    x = x_ref[...]
    h = jax.nn.gelu(_ln(x @ W1_ref[...] + b1_ref[...], g1_ref[...], be1_ref[...]))
    z = _ln(h @ W2_ref[...] + b2_ref[...], g2_ref[...], be2_ref[...])
    z_ref[...] = z
    zrT = lax.dot_general(R_ref[...], z, (((0,), (1,)), ((), ())))
    iota_k = lax.broadcasted_iota(jnp.int32, (K, BB), 0).astype(jnp.float32)
    for hh in range(H):
        zhT = zrT[hh * HD:(hh + 1) * HD, :]
        aT = jnp.sum(zhT * zhT, axis=0, keepdims=True)
        pT = lax.dot_general(cb_ref[hh], zhT, (((1,), (0,)), ((), ())))
        dT = aT - 2.0 * pT + cb2t_ref[:, hh:hh + 1]
        m = jnp.min(dT, axis=0, keepdims=True)
        idxf = jnp.min(jnp.where(dT == m, iota_k, float(K)), axis=0)
        idx_ref[hh, :] = idxf.astype(jnp.int32)


def _dec_body(q_ref, RTp_ref, W3_ref, b3_ref, g3_ref, be3_ref, W4_ref, b4_ref,
              zq_ref, rec_ref):
    qp = q_ref[...].reshape(BB, H * HDP)
    zq = lax.dot_general(qp.astype(jnp.bfloat16), RTp_ref[...],
                         (((1,), (0,)), ((), ())),
                         preferred_element_type=jnp.float32)
    zq_ref[...] = zq
    h2 = jax.nn.gelu(_ln(
        lax.dot_general(zq.astype(jnp.bfloat16), W3_ref[...],
                        (((1,), (0,)), ((), ())),
                        preferred_element_type=jnp.float32) + b3_ref[...],
        g3_ref[...], be3_ref[...]))
    rec_ref[...] = lax.dot_general(h2.astype(jnp.bfloat16), W4_ref[...],
                                   (((1,), (0,)), ((), ())),
                                   preferred_element_type=jnp.float32) + b4_ref[...]


def _full(shape):
    return pl.BlockSpec(shape, lambda i: tuple(0 for _ in shape))


def _encoder_call(x, W1, b1, g1, be1, W2, b2, g2, be2, R, codebook, cb2t, s):
    return pl.pallas_call(
        _enc_body,
        grid=(NBLK,),
        in_specs=[
            pl.BlockSpec((BB, D_IN), lambda i: (i + s * NBLK, 0)),
            _full((D_IN, D_HID)), _full((D_HID,)), _full((D_HID,)), _full((D_HID,)),
            _full((D_HID, D_LAT)), _full((D_LAT,)), _full((D_LAT,)), _full((D_LAT,)),
            _full((D_LAT, D_LAT)),
            _full((H, K, HD)),
            _full((K, H)),
        ],
        out_specs=[
            pl.BlockSpec((BB, D_LAT), lambda i: (i, 0)),
            pl.BlockSpec((H, BB), lambda i: (0, i)),
        ],
        out_shape=[
            jax.ShapeDtypeStruct((CS, D_LAT), jnp.float32),
            jax.ShapeDtypeStruct((H, CS), jnp.int32),
        ],
        compiler_params=pltpu.CompilerParams(
            dimension_semantics=("parallel",)),
    )(x, W1, b1, g1, be1, W2, b2, g2, be2, R, codebook, cb2t)


def _decoder_call(qcat, RTp, W3, b3, g3, be3, W4, b4):
    return pl.pallas_call(
        _dec_body,
        grid=(NBLK,),
        in_specs=[
            pl.BlockSpec((BB * H, HDP), lambda i: (i, 0)),
            _full((H * HDP, D_LAT)),
            _full((D_LAT, D_HID)), _full((D_HID,)), _full((D_HID,)), _full((D_HID,)),
            _full((D_HID, D_IN)), _full((D_IN,)),
        ],
        out_specs=[
            pl.BlockSpec((BB, D_LAT), lambda i: (i, 0)),
            pl.BlockSpec((BB, D_IN), lambda i: (i, 0)),
        ],
        out_shape=[
            jax.ShapeDtypeStruct((CS, D_LAT), jnp.float32),
            jax.ShapeDtypeStruct((CS, D_IN), jnp.float32),
        ],
        compiler_params=pltpu.CompilerParams(
            dimension_semantics=("parallel",)),
    )(qcat, RTp, W3, b3, g3, be3, W4, b4)


TABW = H * K * HD
CH2 = 64


def _sc_gather(table_flat, idx_hb):
    n_b = idx_hb.shape[1]
    BPW = n_b * H // NW
    BBW = n_b // NW
    mesh = plsc.VectorSubcoreMesh(core_axis_name="c", subcore_axis_name="s")

    @functools.partial(
        pl.kernel,
        mesh=mesh,
        out_type=jax.ShapeDtypeStruct((n_b * H, HDP), jnp.float32),
        scratch_types=[
            pltpu.VMEM((TABW,), jnp.float32),
            pltpu.VMEM((H * BBW,), jnp.int32),
            pltpu.VMEM((CH2, HDP), jnp.float32),
            pltpu.VMEM((CH2, HDP), jnp.float32),
            pltpu.SemaphoreType.DMA,
            pltpu.SemaphoreType.DMA,
        ],
    )
    def gather_k(table_hbm, idx_hbm, out_hbm, tab_v, idx_v, buf0, buf1,
                 sem0, sem1):
        wid = lax.axis_index("s") * NC + lax.axis_index("c")
        base = wid * BPW
        b0 = wid * BBW
        pltpu.sync_copy(table_hbm, tab_v)
        for hh in range(H):
            pltpu.sync_copy(idx_hbm.at[hh, pl.ds(b0, BBW)],
                            idx_v.at[pl.ds(hh * BBW, BBW)])
        bufs = (buf0, buf1)
        sems = (sem0, sem1)
        z16 = jnp.zeros((16,), jnp.float32)
        for b in range(2):
            for r in range(CH2):
                bufs[b][r, pl.ds(HD, 16)] = z16
                bufs[b][r, pl.ds(HD + 16, 16)] = z16

        nsup = BPW // (2 * CH2)
        nb_ch = CH2 // H

        def super_body(c2, carry):
            for b in range(2):
                ch = c2 * 2 + b
                row0 = ch * CH2
                bl0 = ch * nb_ch
                g = [idx_v[pl.ds(hh * BBW + bl0, 16)] * HD + hh * (K * HD)
                     for hh in range(H)]
                for r in range(CH2):
                    src = g[r % H][r // H]
                    for c6 in range(HD // 16):
                        bufs[b][r, pl.ds(16 * c6, 16)] = (
                            tab_v[pl.ds(src + 16 * c6, 16)])
                pltpu.async_copy(bufs[b],
                                 out_hbm.at[pl.ds(base + row0, CH2)],
                                 sems[b])
            for b in range(2):
                pltpu.make_async_copy(
                    bufs[b],
                    out_hbm.at[pl.ds(base + (c2 * 2 + b) * CH2, CH2)],
                    sems[b]).wait()
            return carry

        lax.fori_loop(0, nsup, super_body, 0)

    return gather_k(table_flat, idx_hb)


def kernel(x, W1, b1, g1, be1, W2, b2, g2, be2, R, codebook, W3, b3, g3, be3,
           W4, b4):
    cb2t = jnp.sum(codebook * codebook, axis=-1).T
    table = codebook.reshape(-1)
    RTp = jnp.pad(R.T.reshape(H, HD, D_LAT), ((0, 0), (0, HDP - HD), (0, 0))
                  ).reshape(H * HDP, D_LAT).astype(jnp.bfloat16)
    W3b = W3.astype(jnp.bfloat16)
    W4b = W4.astype(jnp.bfloat16)

    zs, idxs, qs = [], [], []
    for s in range(S):
        z_s, idx_s = _encoder_call(x, W1, b1, g1, be1, W2, b2, g2, be2, R,
                                   codebook, cb2t, s)
        qs.append(_sc_gather(table, idx_s))
        zs.append(z_s)
        idxs.append(idx_s)
    outs = [_decoder_call(q, RTp, W3b, b3, g3, be3, W4b, b4) for q in qs]
    reconstructed = jnp.concatenate([o[1] for o in outs], axis=0)
    z_q = jnp.concatenate([o[0] for o in outs], axis=0)
    z = jnp.concatenate(zs, axis=0)
    indices = jnp.concatenate(idxs, axis=1).T
    return (reconstructed, indices, z, z_q)


# speedup vs baseline: 1.6346x; 1.1142x over previous
import functools

import jax
import jax.numpy as jnp
from jax import lax
from jax.experimental import pallas as pl
from jax.experimental.pallas import tpu as pltpu
from jax.experimental.pallas import tpu_sc as plsc

H = 4
K = 256
D_IN = 384
D_HID = 256
D_LAT = 384
HD = D_LAT // H
B = 16384

BB = 512
S = 1
CS = B // S
NBLK = CS // BB

NC = 2
NS = 16
NW = NC * NS
CH = 256
HDP = 128


def _ln(x, g, b):
    mu = jnp.mean(x, axis=-1, keepdims=True)
    var = jnp.var(x, axis=-1, keepdims=True)
    return (x - mu) / jnp.sqrt(var + 1e-5) * g + b


def _enc_body(x_ref, W1_ref, b1_ref, g1_ref, be1_ref, W2_ref, b2_ref,
              g2_ref, be2_ref, R_ref, cb_ref, cb2t_ref, z_ref, idx_ref):
    x = x_ref[...]
    h = jax.nn.gelu(_ln(x @ W1_ref[...] + b1_ref[...], g1_ref[...], be1_ref[...]))
    z = _ln(h @ W2_ref[...] + b2_ref[...], g2_ref[...], be2_ref[...])
    z_ref[...] = z
    zrT = lax.dot_general(R_ref[...], z, (((0,), (1,)), ((), ())))
    iota_k = lax.broadcasted_iota(jnp.int32, (K, BB), 0).astype(jnp.float32)
    for hh in range(H):
        zhT = zrT[hh * HD:(hh + 1) * HD, :]
        aT = jnp.sum(zhT * zhT, axis=0, keepdims=True)
        pT = lax.dot_general(cb_ref[hh], zhT, (((1,), (0,)), ((), ())))
        dT = aT - 2.0 * pT + cb2t_ref[:, hh:hh + 1]
        m = jnp.min(dT, axis=0, keepdims=True)
        idxf = jnp.min(jnp.where(dT == m, iota_k, float(K)), axis=0)
        idx_ref[hh, :] = idxf.astype(jnp.int32)


def _dec_body(q_ref, RTp_ref, W3_ref, b3_ref, g3_ref, be3_ref, W4_ref, b4_ref,
              zq_ref, rec_ref):
    qp = q_ref[...].reshape(BB, H * HDP)
    zq = lax.dot_general(qp.astype(jnp.bfloat16), RTp_ref[...],
                         (((1,), (0,)), ((), ())),
                         preferred_element_type=jnp.float32)
    zq_ref[...] = zq
    h2 = jax.nn.gelu(_ln(
        lax.dot_general(zq.astype(jnp.bfloat16), W3_ref[...],
                        (((1,), (0,)), ((), ())),
                        preferred_element_type=jnp.float32) + b3_ref[...],
        g3_ref[...], be3_ref[...]))
    rec_ref[...] = lax.dot_general(h2.astype(jnp.bfloat16), W4_ref[...],
                                   (((1,), (0,)), ((), ())),
                                   preferred_element_type=jnp.float32) + b4_ref[...]


def _full(shape):
    return pl.BlockSpec(shape, lambda i: tuple(0 for _ in shape))


def _encoder_call(x, W1, b1, g1, be1, W2, b2, g2, be2, R, codebook, cb2t, s):
    return pl.pallas_call(
        _enc_body,
        grid=(NBLK,),
        in_specs=[
            pl.BlockSpec((BB, D_IN), lambda i: (i + s * NBLK, 0)),
            _full((D_IN, D_HID)), _full((D_HID,)), _full((D_HID,)), _full((D_HID,)),
            _full((D_HID, D_LAT)), _full((D_LAT,)), _full((D_LAT,)), _full((D_LAT,)),
            _full((D_LAT, D_LAT)),
            _full((H, K, HD)),
            _full((K, H)),
        ],
        out_specs=[
            pl.BlockSpec((BB, D_LAT), lambda i: (i, 0)),
            pl.BlockSpec((H, BB), lambda i: (0, i)),
        ],
        out_shape=[
            jax.ShapeDtypeStruct((CS, D_LAT), jnp.float32),
            jax.ShapeDtypeStruct((H, CS), jnp.int32),
        ],
        compiler_params=pltpu.CompilerParams(
            dimension_semantics=("parallel",)),
    )(x, W1, b1, g1, be1, W2, b2, g2, be2, R, codebook, cb2t)


def _decoder_call(qcat, RTp, W3, b3, g3, be3, W4, b4):
    return pl.pallas_call(
        _dec_body,
        grid=(NBLK,),
        in_specs=[
            pl.BlockSpec((BB * H, HDP), lambda i: (i, 0)),
            _full((H * HDP, D_LAT)),
            _full((D_LAT, D_HID)), _full((D_HID,)), _full((D_HID,)), _full((D_HID,)),
            _full((D_HID, D_IN)), _full((D_IN,)),
        ],
        out_specs=[
            pl.BlockSpec((BB, D_LAT), lambda i: (i, 0)),
            pl.BlockSpec((BB, D_IN), lambda i: (i, 0)),
        ],
        out_shape=[
            jax.ShapeDtypeStruct((CS, D_LAT), jnp.float32),
            jax.ShapeDtypeStruct((CS, D_IN), jnp.float32),
        ],
        compiler_params=pltpu.CompilerParams(
            dimension_semantics=("parallel",)),
    )(qcat, RTp, W3, b3, g3, be3, W4, b4)


TABW = H * K * HD
CH2 = 64


def _sc_gather(table_flat, idx_hb):
    n_b = idx_hb.shape[1]
    BPW = n_b * H // NW
    BBW = n_b // NW
    mesh = plsc.VectorSubcoreMesh(core_axis_name="c", subcore_axis_name="s")

    @functools.partial(
        pl.kernel,
        mesh=mesh,
        out_type=jax.ShapeDtypeStruct((n_b * H, HDP), jnp.float32),
        scratch_types=[
            pltpu.VMEM((TABW,), jnp.float32),
            pltpu.VMEM((H * BBW,), jnp.int32),
            pltpu.VMEM((CH2, HDP), jnp.float32),
            pltpu.VMEM((CH2, HDP), jnp.float32),
            pltpu.SemaphoreType.DMA,
            pltpu.SemaphoreType.DMA,
        ],
    )
    def gather_k(table_hbm, idx_hbm, out_hbm, tab_v, idx_v, buf0, buf1,
                 sem0, sem1):
        wid = lax.axis_index("s") * NC + lax.axis_index("c")
        base = wid * BPW
        b0 = wid * BBW
        pltpu.sync_copy(table_hbm, tab_v)
        for hh in range(H):
            pltpu.sync_copy(idx_hbm.at[hh, pl.ds(b0, BBW)],
                            idx_v.at[pl.ds(hh * BBW, BBW)])
        bufs = (buf0, buf1)
        sems = (sem0, sem1)
        z16 = jnp.zeros((16,), jnp.float32)
        for b in range(2):
            for r in range(CH2):
                bufs[b][r, pl.ds(HD, 16)] = z16
                bufs[b][r, pl.ds(HD + 16, 16)] = z16

        nsup = BPW // (2 * CH2)
        nb_ch = CH2 // H

        def super_body(c2, carry):
            for b in range(2):
                ch = c2 * 2 + b
                row0 = ch * CH2
                bl0 = ch * nb_ch
                g = [idx_v[pl.ds(hh * BBW + bl0, 16)] * HD + hh * (K * HD)
                     for hh in range(H)]
                for r in range(CH2):
                    src = g[r % H][r // H]
                    vals = [tab_v[pl.ds(src + 16 * c6, 16)]
                            for c6 in range(HD // 16)]
                    for c6 in range(HD // 16):
                        bufs[b][r, pl.ds(16 * c6, 16)] = vals[c6]
                pltpu.async_copy(bufs[b],
                                 out_hbm.at[pl.ds(base + row0, CH2)],
                                 sems[b])
            for b in range(2):
                pltpu.make_async_copy(
                    bufs[b],
                    out_hbm.at[pl.ds(base + (c2 * 2 + b) * CH2, CH2)],
                    sems[b]).wait()
            return carry

        lax.fori_loop(0, nsup, super_body, 0)

    return gather_k(table_flat, idx_hb)


def kernel(x, W1, b1, g1, be1, W2, b2, g2, be2, R, codebook, W3, b3, g3, be3,
           W4, b4):
    cb2t = jnp.sum(codebook * codebook, axis=-1).T
    table = codebook.reshape(-1)
    RTp = jnp.pad(R.T.reshape(H, HD, D_LAT), ((0, 0), (0, HDP - HD), (0, 0))
                  ).reshape(H * HDP, D_LAT).astype(jnp.bfloat16)
    W3b = W3.astype(jnp.bfloat16)
    W4b = W4.astype(jnp.bfloat16)

    zs, idxs, qs = [], [], []
    for s in range(S):
        z_s, idx_s = _encoder_call(x, W1, b1, g1, be1, W2, b2, g2, be2, R,
                                   codebook, cb2t, s)
        qs.append(_sc_gather(table, idx_s))
        zs.append(z_s)
        idxs.append(idx_s)
    outs = [_decoder_call(q, RTp, W3b, b3, g3, be3, W4b, b4) for q in qs]
    reconstructed = jnp.concatenate([o[1] for o in outs], axis=0)
    z_q = jnp.concatenate([o[0] for o in outs], axis=0)
    z = jnp.concatenate(zs, axis=0)
    indices = jnp.concatenate(idxs, axis=1).T
    return (reconstructed, indices, z, z_q)


# speedup vs baseline: 1.6730x; 1.0235x over previous
import functools

import jax
import jax.numpy as jnp
from jax import lax
from jax.experimental import pallas as pl
from jax.experimental.pallas import tpu as pltpu
from jax.experimental.pallas import tpu_sc as plsc

H = 4
K = 256
D_IN = 384
D_HID = 256
D_LAT = 384
HD = D_LAT // H
B = 16384

BB = 512
S = 1
CS = B // S
NBLK = CS // BB

NC = 2
NS = 16
NW = NC * NS
CH = 256
HDP = 128


def _ln(x, g, b):
    mu = jnp.mean(x, axis=-1, keepdims=True)
    var = jnp.var(x, axis=-1, keepdims=True)
    return (x - mu) / jnp.sqrt(var + 1e-5) * g + b


def _lnm(x, g, b, o):
    mu = lax.dot_general(x, o, (((1,), (0,)), ((), ())))[:, 0:1]
    m2 = lax.dot_general(x * x, o, (((1,), (0,)), ((), ())))[:, 0:1]
    var = m2 - mu * mu
    return (x - mu) / jnp.sqrt(var + 1e-5) * g + b


def _enc_body(x_ref, W1_ref, b1_ref, g1_ref, be1_ref, W2_ref, b2_ref,
              g2_ref, be2_ref, R_ref, cb_ref, cb2t_ref, on1_ref, on2_ref,
              z_ref, idx_ref):
    x = x_ref[...]
    h = jax.nn.gelu(_ln(x @ W1_ref[...] + b1_ref[...], g1_ref[...],
                        be1_ref[...]))
    z = _ln(h @ W2_ref[...] + b2_ref[...], g2_ref[...], be2_ref[...])
    z_ref[...] = z
    zrT = lax.dot_general(R_ref[...], z, (((0,), (1,)), ((), ())))
    iota_k = lax.broadcasted_iota(jnp.int32, (K, BB), 0).astype(jnp.float32)
    for hh in range(H):
        zhT = zrT[hh * HD:(hh + 1) * HD, :]
        pT = lax.dot_general(cb_ref[hh], zhT, (((1,), (0,)), ((), ())))
        dT = cb2t_ref[:, hh:hh + 1] - 2.0 * pT
        m = jnp.min(dT, axis=0, keepdims=True)
        idxf = jnp.min(jnp.where(dT == m, iota_k, float(K)), axis=0)
        idx_ref[hh, :] = idxf.astype(jnp.int32)


def _dec_body(q_ref, Rb_ref, W3_ref, b3_ref, g3_ref, be3_ref, W4_ref, b4_ref,
              zq_ref, rec_ref):
    qp = q_ref[...]
    zq = lax.dot_general(qp.astype(jnp.bfloat16), Rb_ref[...],
                         (((1,), (1,)), ((), ())),
                         preferred_element_type=jnp.float32)
    zq_ref[...] = zq
    h2 = jax.nn.gelu(_ln(
        lax.dot_general(zq.astype(jnp.bfloat16), W3_ref[...],
                        (((1,), (0,)), ((), ())),
                        preferred_element_type=jnp.float32) + b3_ref[...],
        g3_ref[...], be3_ref[...]))
    rec_ref[...] = lax.dot_general(h2.astype(jnp.bfloat16), W4_ref[...],
                                   (((1,), (0,)), ((), ())),
                                   preferred_element_type=jnp.float32) + b4_ref[...]


def _full(shape):
    return pl.BlockSpec(shape, lambda i: tuple(0 for _ in shape))


def _encoder_call(x, W1, b1, g1, be1, W2, b2, g2, be2, R, codebook, cb2t,
                  on1, on2, s):
    return pl.pallas_call(
        _enc_body,
        grid=(NBLK,),
        in_specs=[
            pl.BlockSpec((BB, D_IN), lambda i: (i + s * NBLK, 0)),
            _full((D_IN, D_HID)), _full((D_HID,)), _full((D_HID,)), _full((D_HID,)),
            _full((D_HID, D_LAT)), _full((D_LAT,)), _full((D_LAT,)), _full((D_LAT,)),
            _full((D_LAT, D_LAT)),
            _full((H, K, HD)),
            _full((K, H)),
            _full((D_HID, 128)),
            _full((D_LAT, 128)),
        ],
        out_specs=[
            pl.BlockSpec((BB, D_LAT), lambda i: (i, 0)),
            pl.BlockSpec((H, BB), lambda i: (0, i)),
        ],
        out_shape=[
            jax.ShapeDtypeStruct((CS, D_LAT), jnp.float32),
            jax.ShapeDtypeStruct((H, CS), jnp.int32),
        ],
        compiler_params=pltpu.CompilerParams(
            dimension_semantics=("parallel",)),
    )(x, W1, b1, g1, be1, W2, b2, g2, be2, R, codebook, cb2t, on1, on2)


def _decoder_call(qcat, Rb, W3, b3, g3, be3, W4, b4):
    return pl.pallas_call(
        _dec_body,
        grid=(NBLK,),
        in_specs=[
            pl.BlockSpec((BB, D_LAT), lambda i: (i, 0)),
            _full((D_LAT, D_LAT)),
            _full((D_LAT, D_HID)), _full((D_HID,)), _full((D_HID,)), _full((D_HID,)),
            _full((D_HID, D_IN)), _full((D_IN,)),
        ],
        out_specs=[
            pl.BlockSpec((BB, D_LAT), lambda i: (i, 0)),
            pl.BlockSpec((BB, D_IN), lambda i: (i, 0)),
        ],
        out_shape=[
            jax.ShapeDtypeStruct((CS, D_LAT), jnp.float32),
            jax.ShapeDtypeStruct((CS, D_IN), jnp.float32),
        ],
        compiler_params=pltpu.CompilerParams(
            dimension_semantics=("parallel",)),
    )(qcat, Rb, W3, b3, g3, be3, W4, b4)


TABW = H * K * HD
CH2 = 64


def _sc_gather(table_flat, idx_hb):
    n_b = idx_hb.shape[1]
    BPW = n_b * H // NW
    BBW = n_b // NW
    mesh = plsc.VectorSubcoreMesh(core_axis_name="c", subcore_axis_name="s")

    nb_ch = CH2 // H

    @functools.partial(
        pl.kernel,
        mesh=mesh,
        out_type=jax.ShapeDtypeStruct((n_b, D_LAT), jnp.float32),
        scratch_types=[
            pltpu.VMEM((TABW,), jnp.float32),
            pltpu.VMEM((H * BBW,), jnp.int32),
            pltpu.VMEM((nb_ch, D_LAT), jnp.float32),
            pltpu.VMEM((nb_ch, D_LAT), jnp.float32),
            pltpu.SemaphoreType.DMA,
            pltpu.SemaphoreType.DMA,
        ],
    )
    def gather_k(table_hbm, idx_hbm, out_hbm, tab_v, idx_v, buf0, buf1,
                 sem0, sem1):
        wid = lax.axis_index("s") * NC + lax.axis_index("c")
        b0 = wid * BBW
        pltpu.sync_copy(table_hbm, tab_v)
        for hh in range(H):
            pltpu.sync_copy(idx_hbm.at[hh, pl.ds(b0, BBW)],
                            idx_v.at[pl.ds(hh * BBW, BBW)])
        bufs = (buf0, buf1)
        sems = (sem0, sem1)

        nsup = BPW // (2 * CH2)

        def super_body(c2, carry):
            for b in range(2):
                ch = c2 * 2 + b
                bl0 = ch * nb_ch
                g = [idx_v[pl.ds(hh * BBW + bl0, 16)] * HD + hh * (K * HD)
                     for hh in range(H)]
                for r in range(CH2):
                    src = g[r % H][r // H]
                    vals = [tab_v[pl.ds(src + 16 * c6, 16)]
                            for c6 in range(HD // 16)]
                    for c6 in range(HD // 16):
                        bufs[b][r // H, pl.ds((r % H) * HD + 16 * c6, 16)] = (
                            vals[c6])
                pltpu.async_copy(bufs[b],
                                 out_hbm.at[pl.ds(b0 + bl0, nb_ch)],
                                 sems[b])
            for b in range(2):
                pltpu.make_async_copy(
                    bufs[b],
                    out_hbm.at[pl.ds(b0 + (c2 * 2 + b) * nb_ch, nb_ch)],
                    sems[b]).wait()
            return carry

        lax.fori_loop(0, nsup, super_body, 0)

    return gather_k(table_flat, idx_hb)


def kernel(x, W1, b1, g1, be1, W2, b2, g2, be2, R, codebook, W3, b3, g3, be3,
           W4, b4):
    cb2t = jnp.sum(codebook * codebook, axis=-1).T
    table = codebook.reshape(-1)
    W3b = W3.astype(jnp.bfloat16)
    W4b = W4.astype(jnp.bfloat16)
    Rb = R.astype(jnp.bfloat16)
    on1 = jnp.full((D_HID, 128), 1.0 / D_HID, dtype=jnp.float32)
    on2 = jnp.full((D_LAT, 128), 1.0 / D_LAT, dtype=jnp.float32)

    zs, idxs, qs = [], [], []
    for s in range(S):
        z_s, idx_s = _encoder_call(x, W1, b1, g1, be1, W2, b2, g2, be2, R,
                                   codebook, cb2t, on1, on2, s)
        qs.append(_sc_gather(table, idx_s))
        zs.append(z_s)
        idxs.append(idx_s)
    outs = [_decoder_call(q, Rb, W3b, b3, g3, be3, W4b, b4) for q in qs]
    reconstructed = jnp.concatenate([o[1] for o in outs], axis=0)
    z_q = jnp.concatenate([o[0] for o in outs], axis=0)
    z = jnp.concatenate(zs, axis=0)
    indices = jnp.concatenate(idxs, axis=1).T
    return (reconstructed, indices, z, z_q)


# speedup vs baseline: 1.6922x; 1.0115x over previous
import functools

import jax
import jax.numpy as jnp
from jax import lax
from jax.experimental import pallas as pl
from jax.experimental.pallas import tpu as pltpu
from jax.experimental.pallas import tpu_sc as plsc

H = 4
K = 256
D_IN = 384
D_HID = 256
D_LAT = 384
HD = D_LAT // H
B = 16384

BB = 512
S = 1
CS = B // S
NBLK = CS // BB

NC = 2
NS = 16
NW = NC * NS
CH = 256
HDP = 128


def _ln(x, g, b):
    mu = jnp.mean(x, axis=-1, keepdims=True)
    var = jnp.var(x, axis=-1, keepdims=True)
    return (x - mu) / jnp.sqrt(var + 1e-5) * g + b


def _lnm(x, g, b, o):
    mu = lax.dot_general(x, o, (((1,), (0,)), ((), ())))[:, 0:1]
    m2 = lax.dot_general(x * x, o, (((1,), (0,)), ((), ())))[:, 0:1]
    var = m2 - mu * mu
    return (x - mu) / jnp.sqrt(var + 1e-5) * g + b


def _enc_body(x_ref, W1_ref, b1_ref, g1_ref, be1_ref, W2_ref, b2_ref,
              g2_ref, be2_ref, R_ref, cb_ref, cb2t_ref, on1_ref, on2_ref,
              z_ref, idx_ref):
    x = x_ref[...]
    h = jax.nn.gelu(_ln(x @ W1_ref[...] + b1_ref[...], g1_ref[...],
                        be1_ref[...]))
    z = _ln(h @ W2_ref[...] + b2_ref[...], g2_ref[...], be2_ref[...])
    z_ref[...] = z
    zrT = lax.dot_general(R_ref[...], z, (((0,), (1,)), ((), ())))
    iota_k = lax.broadcasted_iota(jnp.int32, (K, BB), 0).astype(jnp.float32)
    for hh in range(H):
        zhT = zrT[hh * HD:(hh + 1) * HD, :]
        pT = lax.dot_general(cb_ref[hh], zhT, (((1,), (0,)), ((), ())))
        dT = cb2t_ref[:, hh:hh + 1] - 2.0 * pT
        m = jnp.min(dT, axis=0, keepdims=True)
        idxf = jnp.min(jnp.where(dT == m, iota_k, float(K)), axis=0)
        idx_ref[hh, :] = idxf.astype(jnp.int32)


def _dec_body(q_ref, Rb_ref, W3_ref, b3_ref, g3_ref, be3_ref, W4_ref, b4_ref,
              rec_ref):
    qp = q_ref[...]
    zq = lax.dot_general(qp.astype(jnp.bfloat16), Rb_ref[...],
                         (((1,), (1,)), ((), ())),
                         preferred_element_type=jnp.float32)
    h2 = jax.nn.gelu(_ln(
        lax.dot_general(zq.astype(jnp.bfloat16), W3_ref[...],
                        (((1,), (0,)), ((), ())),
                        preferred_element_type=jnp.float32) + b3_ref[...],
        g3_ref[...], be3_ref[...]))
    rec_ref[...] = lax.dot_general(h2.astype(jnp.bfloat16), W4_ref[...],
                                   (((1,), (0,)), ((), ())),
                                   preferred_element_type=jnp.float32) + b4_ref[...]


def _full(shape):
    return pl.BlockSpec(shape, lambda i: tuple(0 for _ in shape))


def _encoder_call(x, W1, b1, g1, be1, W2, b2, g2, be2, R, codebook, cb2t,
                  on1, on2, s):
    return pl.pallas_call(
        _enc_body,
        grid=(NBLK,),
        in_specs=[
            pl.BlockSpec((BB, D_IN), lambda i: (i + s * NBLK, 0)),
            _full((D_IN, D_HID)), _full((D_HID,)), _full((D_HID,)), _full((D_HID,)),
            _full((D_HID, D_LAT)), _full((D_LAT,)), _full((D_LAT,)), _full((D_LAT,)),
            _full((D_LAT, D_LAT)),
            _full((H, K, HD)),
            _full((K, H)),
            _full((D_HID, 128)),
            _full((D_LAT, 128)),
        ],
        out_specs=[
            pl.BlockSpec((BB, D_LAT), lambda i: (i, 0)),
            pl.BlockSpec((H, BB), lambda i: (0, i)),
        ],
        out_shape=[
            jax.ShapeDtypeStruct((CS, D_LAT), jnp.float32),
            jax.ShapeDtypeStruct((H, CS), jnp.int32),
        ],
        compiler_params=pltpu.CompilerParams(
            dimension_semantics=("parallel",)),
    )(x, W1, b1, g1, be1, W2, b2, g2, be2, R, codebook, cb2t, on1, on2)


def _decoder_call(qcat, Rb, W3, b3, g3, be3, W4, b4):
    return pl.pallas_call(
        _dec_body,
        grid=(NBLK,),
        in_specs=[
            pl.BlockSpec((BB, D_LAT), lambda i: (i, 0)),
            _full((D_LAT, D_LAT)),
            _full((D_LAT, D_HID)), _full((D_HID,)), _full((D_HID,)), _full((D_HID,)),
            _full((D_HID, D_IN)), _full((D_IN,)),
        ],
        out_specs=[
            pl.BlockSpec((BB, D_IN), lambda i: (i, 0)),
        ],
        out_shape=[
            jax.ShapeDtypeStruct((CS, D_IN), jnp.float32),
        ],
        compiler_params=pltpu.CompilerParams(
            dimension_semantics=("parallel",)),
    )(qcat, Rb, W3, b3, g3, be3, W4, b4)


TABW = H * K * HD
CH2 = 64


def _sc_gather(table_flat, idx_hb):
    n_b = idx_hb.shape[1]
    BPW = n_b * H // NW
    BBW = n_b // NW
    mesh = plsc.VectorSubcoreMesh(core_axis_name="c", subcore_axis_name="s")

    nb_ch = CH2 // H

    @functools.partial(
        pl.kernel,
        mesh=mesh,
        out_type=jax.ShapeDtypeStruct((n_b, D_LAT), jnp.float32),
        scratch_types=[
            pltpu.VMEM((TABW,), jnp.float32),
            pltpu.VMEM((H * BBW,), jnp.int32),
            pltpu.VMEM((nb_ch, D_LAT), jnp.float32),
            pltpu.VMEM((nb_ch, D_LAT), jnp.float32),
            pltpu.SemaphoreType.DMA,
            pltpu.SemaphoreType.DMA,
        ],
    )
    def gather_k(table_hbm, idx_hbm, out_hbm, tab_v, idx_v, buf0, buf1,
                 sem0, sem1):
        wid = lax.axis_index("s") * NC + lax.axis_index("c")
        b0 = wid * BBW
        pltpu.sync_copy(table_hbm, tab_v)
        for hh in range(H):
            pltpu.sync_copy(idx_hbm.at[hh, pl.ds(b0, BBW)],
                            idx_v.at[pl.ds(hh * BBW, BBW)])
        bufs = (buf0, buf1)
        sems = (sem0, sem1)

        nsup = BPW // (2 * CH2)

        def super_body(c2, carry):
            for b in range(2):
                ch = c2 * 2 + b
                bl0 = ch * nb_ch
                g = [idx_v[pl.ds(hh * BBW + bl0, 16)] * HD + hh * (K * HD)
                     for hh in range(H)]
                for r in range(CH2):
                    src = g[r % H][r // H]
                    vals = [tab_v[pl.ds(src + 16 * c6, 16)]
                            for c6 in range(HD // 16)]
                    for c6 in range(HD // 16):
                        bufs[b][r // H, pl.ds((r % H) * HD + 16 * c6, 16)] = (
                            vals[c6])
                pltpu.async_copy(bufs[b],
                                 out_hbm.at[pl.ds(b0 + bl0, nb_ch)],
                                 sems[b])
            for b in range(2):
                pltpu.make_async_copy(
                    bufs[b],
                    out_hbm.at[pl.ds(b0 + (c2 * 2 + b) * nb_ch, nb_ch)],
                    sems[b]).wait()
            return carry

        lax.fori_loop(0, nsup, super_body, 0)

    return gather_k(table_flat, idx_hb)


def kernel(x, W1, b1, g1, be1, W2, b2, g2, be2, R, codebook, W3, b3, g3, be3,
           W4, b4):
    cb2t = jnp.sum(codebook * codebook, axis=-1).T
    table = codebook.reshape(-1)
    W3b = W3.astype(jnp.bfloat16)
    W4b = W4.astype(jnp.bfloat16)
    Rb = R.astype(jnp.bfloat16)
    on1 = jnp.full((D_HID, 128), 1.0 / D_HID, dtype=jnp.float32)
    on2 = jnp.full((D_LAT, 128), 1.0 / D_LAT, dtype=jnp.float32)

    zs, idxs, qs = [], [], []
    for s in range(S):
        z_s, idx_s = _encoder_call(x, W1, b1, g1, be1, W2, b2, g2, be2, R,
                                   codebook, cb2t, on1, on2, s)
        qs.append(_sc_gather(table, idx_s))
        zs.append(z_s)
        idxs.append(idx_s)
    outs = [_decoder_call(q, Rb, W3b, b3, g3, be3, W4b, b4) for q in qs]
    reconstructed = jnp.concatenate([o[0] for o in outs], axis=0)
    z_q = jnp.concatenate(qs, axis=0)
    z = jnp.concatenate(zs, axis=0)
    indices = jnp.concatenate(idxs, axis=1).T
    return (reconstructed, indices, z, z_q)


# speedup vs baseline: 1.9032x; 1.1247x over previous
import functools

import jax
import jax.numpy as jnp
from jax import lax
from jax.experimental import pallas as pl
from jax.experimental.pallas import tpu as pltpu
from jax.experimental.pallas import tpu_sc as plsc

H = 4
K = 256
D_IN = 384
D_HID = 256
D_LAT = 384
HD = D_LAT // H
B = 16384

BB = 1024
S = 1
CS = B // S
NBLK = CS // BB

NC = 2
NS = 16
NW = NC * NS
CH = 256
HDP = 128


def _ln(x, g, b):
    mu = jnp.mean(x, axis=-1, keepdims=True)
    var = jnp.var(x, axis=-1, keepdims=True)
    return (x - mu) / jnp.sqrt(var + 1e-5) * g + b


def _lnm(x, g, b, o):
    mu = lax.dot_general(x, o, (((1,), (0,)), ((), ())))[:, 0:1]
    m2 = lax.dot_general(x * x, o, (((1,), (0,)), ((), ())))[:, 0:1]
    var = m2 - mu * mu
    return (x - mu) / jnp.sqrt(var + 1e-5) * g + b


def _enc_body(x_ref, W1_ref, b1_ref, g1_ref, be1_ref, W2_ref, b2_ref,
              g2_ref, be2_ref, R_ref, cb_ref, cb2t_ref, on1_ref, on2_ref,
              z_ref, idx_ref):
    x = x_ref[...]
    h = jax.nn.gelu(_ln(x @ W1_ref[...] + b1_ref[...], g1_ref[...],
                        be1_ref[...]))
    z = _ln(h @ W2_ref[...] + b2_ref[...], g2_ref[...], be2_ref[...])
    z_ref[...] = z
    zrT = lax.dot_general(R_ref[...], z, (((0,), (1,)), ((), ())))
    iota_k = lax.broadcasted_iota(jnp.int32, (K, BB), 0).astype(jnp.float32)
    for hh in range(H):
        zhT = zrT[hh * HD:(hh + 1) * HD, :]
        pT = lax.dot_general(cb_ref[hh], zhT, (((1,), (0,)), ((), ())))
        dT = cb2t_ref[:, hh:hh + 1] - 2.0 * pT
        m = jnp.min(dT, axis=0, keepdims=True)
        idxf = jnp.min(jnp.where(dT == m, iota_k, float(K)), axis=0)
        idx_ref[hh, :] = idxf.astype(jnp.int32)


def _dec_body(q_ref, Rb_ref, W3_ref, b3_ref, g3_ref, be3_ref, W4_ref, b4_ref,
              zq_ref, rec_ref):
    qp = q_ref[...]
    zq = lax.dot_general(qp.astype(jnp.bfloat16), Rb_ref[...],
                         (((1,), (1,)), ((), ())),
                         preferred_element_type=jnp.float32)
    zq_ref[...] = zq
    h2 = jax.nn.gelu(_ln(
        lax.dot_general(zq.astype(jnp.bfloat16), W3_ref[...],
                        (((1,), (0,)), ((), ())),
                        preferred_element_type=jnp.float32) + b3_ref[...],
        g3_ref[...], be3_ref[...]))
    rec_ref[...] = lax.dot_general(h2.astype(jnp.bfloat16), W4_ref[...],
                                   (((1,), (0,)), ((), ())),
                                   preferred_element_type=jnp.float32) + b4_ref[...]


def _full(shape):
    return pl.BlockSpec(shape, lambda i: tuple(0 for _ in shape))


def _encoder_call(x, W1, b1, g1, be1, W2, b2, g2, be2, R, codebook, cb2t,
                  on1, on2, s):
    return pl.pallas_call(
        _enc_body,
        grid=(NBLK,),
        in_specs=[
            pl.BlockSpec((BB, D_IN), lambda i: (i + s * NBLK, 0)),
            _full((D_IN, D_HID)), _full((D_HID,)), _full((D_HID,)), _full((D_HID,)),
            _full((D_HID, D_LAT)), _full((D_LAT,)), _full((D_LAT,)), _full((D_LAT,)),
            _full((D_LAT, D_LAT)),
            _full((H, K, HD)),
            _full((K, H)),
            _full((D_HID, 128)),
            _full((D_LAT, 128)),
        ],
        out_specs=[
            pl.BlockSpec((BB, D_LAT), lambda i: (i, 0)),
            pl.BlockSpec((H, BB), lambda i: (0, i)),
        ],
        out_shape=[
            jax.ShapeDtypeStruct((CS, D_LAT), jnp.float32),
            jax.ShapeDtypeStruct((H, CS), jnp.int32),
        ],
        compiler_params=pltpu.CompilerParams(
            dimension_semantics=("parallel",)),
    )(x, W1, b1, g1, be1, W2, b2, g2, be2, R, codebook, cb2t, on1, on2)


def _decoder_call(qcat, Rb, W3, b3, g3, be3, W4, b4):
    return pl.pallas_call(
        _dec_body,
        grid=(NBLK,),
        in_specs=[
            pl.BlockSpec((BB, D_LAT), lambda i: (i, 0)),
            _full((D_LAT, D_LAT)),
            _full((D_LAT, D_HID)), _full((D_HID,)), _full((D_HID,)), _full((D_HID,)),
            _full((D_HID, D_IN)), _full((D_IN,)),
        ],
        out_specs=[
            pl.BlockSpec((BB, D_LAT), lambda i: (i, 0)),
            pl.BlockSpec((BB, D_IN), lambda i: (i, 0)),
        ],
        out_shape=[
            jax.ShapeDtypeStruct((CS, D_LAT), jnp.float32),
            jax.ShapeDtypeStruct((CS, D_IN), jnp.float32),
        ],
        compiler_params=pltpu.CompilerParams(
            dimension_semantics=("parallel",)),
    )(qcat, Rb, W3, b3, g3, be3, W4, b4)


TABW = H * K * HD
CH2 = 64


def _sc_gather(table_flat, idx_hb):
    n_b = idx_hb.shape[1]
    BPW = n_b * H // NW
    BBW = n_b // NW
    mesh = plsc.VectorSubcoreMesh(core_axis_name="c", subcore_axis_name="s")

    nb_ch = CH2 // H

    @functools.partial(
        pl.kernel,
        mesh=mesh,
        out_type=jax.ShapeDtypeStruct((n_b, D_LAT), jnp.float32),
        scratch_types=[
            pltpu.VMEM((TABW,), jnp.float32),
            pltpu.VMEM((H * BBW,), jnp.int32),
            pltpu.VMEM((nb_ch, D_LAT), jnp.float32),
            pltpu.VMEM((nb_ch, D_LAT), jnp.float32),
            pltpu.SemaphoreType.DMA,
            pltpu.SemaphoreType.DMA,
        ],
    )
    def gather_k(table_hbm, idx_hbm, out_hbm, tab_v, idx_v, buf0, buf1,
                 sem0, sem1):
        wid = lax.axis_index("s") * NC + lax.axis_index("c")
        b0 = wid * BBW
        pltpu.sync_copy(table_hbm, tab_v)
        for hh in range(H):
            pltpu.sync_copy(idx_hbm.at[hh, pl.ds(b0, BBW)],
                            idx_v.at[pl.ds(hh * BBW, BBW)])
        bufs = (buf0, buf1)
        sems = (sem0, sem1)

        nsup = BPW // (2 * CH2)

        def super_body(c2, carry):
            for b in range(2):
                ch = c2 * 2 + b
                bl0 = ch * nb_ch
                g = [idx_v[pl.ds(hh * BBW + bl0, 16)] * HD + hh * (K * HD)
                     for hh in range(H)]
                for r in range(CH2):
                    src = g[r % H][r // H]
                    vals = [tab_v[pl.ds(src + 16 * c6, 16)]
                            for c6 in range(HD // 16)]
                    for c6 in range(HD // 16):
                        bufs[b][r // H, pl.ds((r % H) * HD + 16 * c6, 16)] = (
                            vals[c6])
                pltpu.async_copy(bufs[b],
                                 out_hbm.at[pl.ds(b0 + bl0, nb_ch)],
                                 sems[b])
            for b in range(2):
                pltpu.make_async_copy(
                    bufs[b],
                    out_hbm.at[pl.ds(b0 + (c2 * 2 + b) * nb_ch, nb_ch)],
                    sems[b]).wait()
            return carry

        lax.fori_loop(0, nsup, super_body, 0)

    return gather_k(table_flat, idx_hb)


def kernel(x, W1, b1, g1, be1, W2, b2, g2, be2, R, codebook, W3, b3, g3, be3,
           W4, b4):
    cb2t = jnp.sum(codebook * codebook, axis=-1).T
    table = codebook.reshape(-1)
    W3b = W3.astype(jnp.bfloat16)
    W4b = W4.astype(jnp.bfloat16)
    Rb = R.astype(jnp.bfloat16)
    on1 = jnp.full((D_HID, 128), 1.0 / D_HID, dtype=jnp.float32)
    on2 = jnp.full((D_LAT, 128), 1.0 / D_LAT, dtype=jnp.float32)

    zs, idxs, qs = [], [], []
    for s in range(S):
        z_s, idx_s = _encoder_call(x, W1, b1, g1, be1, W2, b2, g2, be2, R,
                                   codebook, cb2t, on1, on2, s)
        qs.append(_sc_gather(table, idx_s))
        zs.append(z_s)
        idxs.append(idx_s)
    outs = [_decoder_call(q, Rb, W3b, b3, g3, be3, W4b, b4) for q in qs]
    reconstructed = jnp.concatenate([o[1] for o in outs], axis=0)
    z_q = jnp.concatenate([o[0] for o in outs], axis=0)
    z = jnp.concatenate(zs, axis=0)
    indices = jnp.concatenate(idxs, axis=1).T
    return (reconstructed, indices, z, z_q)


# speedup vs baseline: 2.0418x; 1.0728x over previous
import functools

import jax
import jax.numpy as jnp
from jax import lax
from jax.experimental import pallas as pl
from jax.experimental.pallas import tpu as pltpu
from jax.experimental.pallas import tpu_sc as plsc

H = 4
K = 256
D_IN = 384
D_HID = 256
D_LAT = 384
HD = D_LAT // H
B = 16384

BB = 2048
S = 1
CS = B // S
NBLK = CS // BB

NC = 2
NS = 16
NW = NC * NS
CH = 256
HDP = 128


def _ln(x, g, b):
    mu = jnp.mean(x, axis=-1, keepdims=True)
    var = jnp.var(x, axis=-1, keepdims=True)
    return (x - mu) / jnp.sqrt(var + 1e-5) * g + b


def _lnm(x, g, b, o):
    mu = lax.dot_general(x, o, (((1,), (0,)), ((), ())))[:, 0:1]
    m2 = lax.dot_general(x * x, o, (((1,), (0,)), ((), ())))[:, 0:1]
    var = m2 - mu * mu
    return (x - mu) / jnp.sqrt(var + 1e-5) * g + b


def _enc_body(x_ref, W1_ref, b1_ref, g1_ref, be1_ref, W2_ref, b2_ref,
              g2_ref, be2_ref, R_ref, cb_ref, cb2t_ref, on1_ref, on2_ref,
              z_ref, idx_ref):
    x = x_ref[...]
    h = jax.nn.gelu(_ln(x @ W1_ref[...] + b1_ref[...], g1_ref[...],
                        be1_ref[...]))
    z = _ln(h @ W2_ref[...] + b2_ref[...], g2_ref[...], be2_ref[...])
    z_ref[...] = z
    zrT = lax.dot_general(R_ref[...], z, (((0,), (1,)), ((), ())))
    iota_k = lax.broadcasted_iota(jnp.int32, (K, BB), 0).astype(jnp.float32)
    for hh in range(H):
        zhT = zrT[hh * HD:(hh + 1) * HD, :]
        pT = lax.dot_general(cb_ref[hh], zhT, (((1,), (0,)), ((), ())))
        dT = cb2t_ref[:, hh:hh + 1] - 2.0 * pT
        m = jnp.min(dT, axis=0, keepdims=True)
        idxf = jnp.min(jnp.where(dT == m, iota_k, float(K)), axis=0)
        idx_ref[hh, :] = idxf.astype(jnp.int32)


def _dec_body(q_ref, Rb_ref, W3_ref, b3_ref, g3_ref, be3_ref, W4_ref, b4_ref,
              zq_ref, rec_ref):
    qp = q_ref[...]
    zq = lax.dot_general(qp.astype(jnp.bfloat16), Rb_ref[...],
                         (((1,), (1,)), ((), ())),
                         preferred_element_type=jnp.float32)
    zq_ref[...] = zq
    h2 = jax.nn.gelu(_ln(
        lax.dot_general(zq.astype(jnp.bfloat16), W3_ref[...],
                        (((1,), (0,)), ((), ())),
                        preferred_element_type=jnp.float32) + b3_ref[...],
        g3_ref[...], be3_ref[...]))
    rec_ref[...] = lax.dot_general(h2.astype(jnp.bfloat16), W4_ref[...],
                                   (((1,), (0,)), ((), ())),
                                   preferred_element_type=jnp.float32) + b4_ref[...]


def _full(shape):
    return pl.BlockSpec(shape, lambda i: tuple(0 for _ in shape))


def _encoder_call(x, W1, b1, g1, be1, W2, b2, g2, be2, R, codebook, cb2t,
                  on1, on2, s):
    return pl.pallas_call(
        _enc_body,
        grid=(NBLK,),
        in_specs=[
            pl.BlockSpec((BB, D_IN), lambda i: (i + s * NBLK, 0)),
            _full((D_IN, D_HID)), _full((D_HID,)), _full((D_HID,)), _full((D_HID,)),
            _full((D_HID, D_LAT)), _full((D_LAT,)), _full((D_LAT,)), _full((D_LAT,)),
            _full((D_LAT, D_LAT)),
            _full((H, K, HD)),
            _full((K, H)),
            _full((D_HID, 128)),
            _full((D_LAT, 128)),
        ],
        out_specs=[
            pl.BlockSpec((BB, D_LAT), lambda i: (i, 0)),
            pl.BlockSpec((H, BB), lambda i: (0, i)),
        ],
        out_shape=[
            jax.ShapeDtypeStruct((CS, D_LAT), jnp.float32),
            jax.ShapeDtypeStruct((H, CS), jnp.int32),
        ],
        compiler_params=pltpu.CompilerParams(
            dimension_semantics=("parallel",)),
    )(x, W1, b1, g1, be1, W2, b2, g2, be2, R, codebook, cb2t, on1, on2)


def _decoder_call(qcat, Rb, W3, b3, g3, be3, W4, b4):
    return pl.pallas_call(
        _dec_body,
        grid=(NBLK,),
        in_specs=[
            pl.BlockSpec((BB, D_LAT), lambda i: (i, 0)),
            _full((D_LAT, D_LAT)),
            _full((D_LAT, D_HID)), _full((D_HID,)), _full((D_HID,)), _full((D_HID,)),
            _full((D_HID, D_IN)), _full((D_IN,)),
        ],
        out_specs=[
            pl.BlockSpec((BB, D_LAT), lambda i: (i, 0)),
            pl.BlockSpec((BB, D_IN), lambda i: (i, 0)),
        ],
        out_shape=[
            jax.ShapeDtypeStruct((CS, D_LAT), jnp.float32),
            jax.ShapeDtypeStruct((CS, D_IN), jnp.float32),
        ],
        compiler_params=pltpu.CompilerParams(
            dimension_semantics=("parallel",)),
    )(qcat, Rb, W3, b3, g3, be3, W4, b4)


TABW = H * K * HD
CH2 = 64


def _sc_gather(table_flat, idx_hb):
    n_b = idx_hb.shape[1]
    BPW = n_b * H // NW
    BBW = n_b // NW
    mesh = plsc.VectorSubcoreMesh(core_axis_name="c", subcore_axis_name="s")

    nb_ch = CH2 // H

    @functools.partial(
        pl.kernel,
        mesh=mesh,
        out_type=jax.ShapeDtypeStruct((n_b, D_LAT), jnp.float32),
        scratch_types=[
            pltpu.VMEM((TABW,), jnp.float32),
            pltpu.VMEM((H * BBW,), jnp.int32),
            pltpu.VMEM((nb_ch, D_LAT), jnp.float32),
            pltpu.VMEM((nb_ch, D_LAT), jnp.float32),
            pltpu.SemaphoreType.DMA,
            pltpu.SemaphoreType.DMA,
        ],
    )
    def gather_k(table_hbm, idx_hbm, out_hbm, tab_v, idx_v, buf0, buf1,
                 sem0, sem1):
        wid = lax.axis_index("s") * NC + lax.axis_index("c")
        b0 = wid * BBW
        pltpu.sync_copy(table_hbm, tab_v)
        for hh in range(H):
            pltpu.sync_copy(idx_hbm.at[hh, pl.ds(b0, BBW)],
                            idx_v.at[pl.ds(hh * BBW, BBW)])
        bufs = (buf0, buf1)
        sems = (sem0, sem1)

        nsup = BPW // (2 * CH2)

        def super_body(c2, carry):
            for b in range(2):
                ch = c2 * 2 + b
                bl0 = ch * nb_ch
                g = [idx_v[pl.ds(hh * BBW + bl0, 16)] * HD + hh * (K * HD)
                     for hh in range(H)]
                for r in range(CH2):
                    src = g[r % H][r // H]
                    vals = [tab_v[pl.ds(src + 16 * c6, 16)]
                            for c6 in range(HD // 16)]
                    for c6 in range(HD // 16):
                        bufs[b][r // H, pl.ds((r % H) * HD + 16 * c6, 16)] = (
                            vals[c6])
                pltpu.async_copy(bufs[b],
                                 out_hbm.at[pl.ds(b0 + bl0, nb_ch)],
                                 sems[b])
            for b in range(2):
                pltpu.make_async_copy(
                    bufs[b],
                    out_hbm.at[pl.ds(b0 + (c2 * 2 + b) * nb_ch, nb_ch)],
                    sems[b]).wait()
            return carry

        lax.fori_loop(0, nsup, super_body, 0)

    return gather_k(table_flat, idx_hb)


def kernel(x, W1, b1, g1, be1, W2, b2, g2, be2, R, codebook, W3, b3, g3, be3,
           W4, b4):
    cb2t = jnp.sum(codebook * codebook, axis=-1).T
    table = codebook.reshape(-1)
    W3b = W3.astype(jnp.bfloat16)
    W4b = W4.astype(jnp.bfloat16)
    Rb = R.astype(jnp.bfloat16)
    on1 = jnp.full((D_HID, 128), 1.0 / D_HID, dtype=jnp.float32)
    on2 = jnp.full((D_LAT, 128), 1.0 / D_LAT, dtype=jnp.float32)

    zs, idxs, qs = [], [], []
    for s in range(S):
        z_s, idx_s = _encoder_call(x, W1, b1, g1, be1, W2, b2, g2, be2, R,
                                   codebook, cb2t, on1, on2, s)
        qs.append(_sc_gather(table, idx_s))
        zs.append(z_s)
        idxs.append(idx_s)
    outs = [_decoder_call(q, Rb, W3b, b3, g3, be3, W4b, b4) for q in qs]
    reconstructed = jnp.concatenate([o[1] for o in outs], axis=0)
    z_q = jnp.concatenate([o[0] for o in outs], axis=0)
    z = jnp.concatenate(zs, axis=0)
    indices = jnp.concatenate(idxs, axis=1).T
    return (reconstructed, indices, z, z_q)
